# baseline (speedup 1.0000x reference)
import jax
import jax.numpy as jnp
from jax import lax
from jax.experimental import pallas as pl
from jax.experimental.pallas import tpu as pltpu
from jax.experimental.pallas import tpu_sc as plsc

NC = 2
NS = 16
NW = NC * NS
D = 128
CHUNK = 64
L = 16
NBUF = 10
GDEPTH = 3


def _sc_embed(ids3, mask3, table):
    K = ids3.shape[1]
    assert K % NBUF == 0
    mesh = plsc.VectorSubcoreMesh(core_axis_name="c", subcore_axis_name="s")

    @pl.kernel(
        out_type=jax.ShapeDtypeStruct((NW, K, CHUNK, D), jnp.float32),
        mesh=mesh,
        scratch_types=[
            pltpu.VMEM((K, CHUNK), jnp.int32),
            pltpu.VMEM((K, CHUNK), jnp.float32),
        ]
        + [pltpu.VMEM((CHUNK, D), jnp.float32) for _ in range(NBUF)]
        + [pltpu.SemaphoreType.DMA for _ in range(2 * NBUF)],
        compiler_params=pltpu.CompilerParams(needs_layout_passes=False),
    )
    def k(ids_hbm, mask_hbm, table_hbm, out_hbm, idx_v, msk_v, *bufs_sems):
        bufs = bufs_sems[:NBUF]
        gsems = bufs_sems[NBUF:2 * NBUF]
        wsems = bufs_sems[2 * NBUF:]
        wid = lax.axis_index("s") * NC + lax.axis_index("c")
        pltpu.sync_copy(ids_hbm.at[wid], idx_v)
        pltpu.sync_copy(mask_hbm.at[wid], msk_v)

        def gather(j, b):
            pltpu.async_copy(table_hbm.at[idx_v.at[j]], bufs[b], gsems[b])

        def wait_gather(j, b):
            pltpu.make_async_copy(
                table_hbm.at[idx_v.at[j]], bufs[b], gsems[b]).wait()

        def write(j, b):
            pltpu.async_copy(bufs[b], out_hbm.at[wid, j], wsems[b])

        def wait_write(j, b):
            pltpu.make_async_copy(
                bufs[b], out_hbm.at[wid, j], wsems[b]).wait()

        def fixup(j, buf):
            def per_group(g, _):
                iv = idx_v[j, pl.ds(g * L, L)]
                mv = msk_v[j, pl.ds(g * L, L)]
                sv = jnp.where(iv == 0, 0.0, mv)
                needs = jnp.any(sv != 1.0)

                @pl.when(needs)
                def _():
                    lanes = lax.iota(jnp.int32, L)

                    def per_lane(l, _):
                        s_s = jnp.sum(jnp.where(lanes == l, sv, 0.0))

                        @pl.when(s_s != 1.0)
                        def _():
                            p = g * L + l
                            for h in range(D // L):
                                sl = pl.ds(h * L, L)
                                buf[p, sl] = buf[p, sl] * s_s
                        return 0

                    lax.fori_loop(0, L, per_lane, 0)
                return 0

            lax.fori_loop(0, CHUNK // L, per_group, 0)

        for j in range(GDEPTH):
            gather(j, j)

        def step(it, _):
            for b in range(NBUF):
                j = it * NBUF + b
                bg = (b + GDEPTH) % NBUF

                @pl.when(jnp.logical_and(j >= NBUF - GDEPTH, j + GDEPTH < K))
                def _():
                    wait_write(j + GDEPTH - NBUF, bg)

                @pl.when(j + GDEPTH < K)
                def _():
                    gather(j + GDEPTH, bg)

                wait_gather(j, b)
                fixup(j, bufs[b])
                write(j, b)
            return 0

        lax.fori_loop(0, K // NBUF, step, 0)

        for j in range(K - NBUF, K):
            wait_write(j, j % NBUF)

    return k(ids3, mask3, table)


def kernel(input_ids, attention_mask, table):
    B, S = input_ids.shape
    n = B * S
    K = n // (NW * CHUNK)
    ids3 = input_ids.reshape(NW, K, CHUNK)
    mask3 = attention_mask.astype(jnp.float32).reshape(NW, K, CHUNK)
    out = _sc_embed(ids3, mask3, table)
    return out.reshape(B, S, D)


# speedup vs baseline: 1.0263x; 1.0263x over previous
import jax
import jax.numpy as jnp
from jax import lax
from jax.experimental import pallas as pl
from jax.experimental.pallas import tpu as pltpu
from jax.experimental.pallas import tpu_sc as plsc

NC = 2
NS = 16
NW = NC * NS
D = 128
CHUNK = 128
L = 16
NBUF = 5
GDEPTH = 2


def _sc_embed(ids3, mask3, table):
    K = ids3.shape[1]
    assert K % NBUF == 0
    mesh = plsc.VectorSubcoreMesh(core_axis_name="c", subcore_axis_name="s")

    @pl.kernel(
        out_type=jax.ShapeDtypeStruct((NW, K, CHUNK, D), jnp.float32),
        mesh=mesh,
        scratch_types=[
            pltpu.VMEM((K, CHUNK), jnp.int32),
            pltpu.VMEM((K, CHUNK), jnp.float32),
        ]
        + [pltpu.VMEM((CHUNK, D), jnp.float32) for _ in range(NBUF)]
        + [pltpu.SemaphoreType.DMA for _ in range(2 * NBUF + 1)],
        compiler_params=pltpu.CompilerParams(needs_layout_passes=False),
    )
    def k(ids_hbm, mask_hbm, table_hbm, out_hbm, idx_v, msk_v, *bufs_sems):
        bufs = bufs_sems[:NBUF]
        gsems = bufs_sems[NBUF:2 * NBUF]
        wsems = bufs_sems[2 * NBUF:3 * NBUF]
        msem = bufs_sems[3 * NBUF]
        wid = lax.axis_index("s") * NC + lax.axis_index("c")
        pltpu.sync_copy(ids_hbm.at[wid], idx_v)
        pltpu.async_copy(mask_hbm.at[wid], msk_v, msem)

        def gather(j, b):
            pltpu.async_copy(table_hbm.at[idx_v.at[j]], bufs[b], gsems[b])

        def wait_gather(j, b):
            pltpu.make_async_copy(
                table_hbm.at[idx_v.at[j]], bufs[b], gsems[b]).wait()

        def write(j, b):
            pltpu.async_copy(bufs[b], out_hbm.at[wid, j], wsems[b])

        def wait_write(j, b):
            pltpu.make_async_copy(
                bufs[b], out_hbm.at[wid, j], wsems[b]).wait()

        def fixup(j, buf):
            def per_group(g, _):
                iv = idx_v[j, pl.ds(g * L, L)]
                mv = msk_v[j, pl.ds(g * L, L)]
                sv = jnp.where(iv == 0, 0.0, mv)
                needs = jnp.any(sv != 1.0)

                @pl.when(needs)
                def _():
                    lanes = lax.iota(jnp.int32, L)

                    def per_lane(l, _):
                        s_s = jnp.sum(jnp.where(lanes == l, sv, 0.0))

                        @pl.when(s_s != 1.0)
                        def _():
                            p = g * L + l
                            for h in range(D // L):
                                sl = pl.ds(h * L, L)
                                buf[p, sl] = buf[p, sl] * s_s
                        return 0

                    lax.fori_loop(0, L, per_lane, 0)
                return 0

            lax.fori_loop(0, CHUNK // L, per_group, 0)

        for j in range(GDEPTH):
            gather(j, j)
        pltpu.make_async_copy(mask_hbm.at[wid], msk_v, msem).wait()

        def step(it, _):
            for b in range(NBUF):
                j = it * NBUF + b
                bg = (b + GDEPTH) % NBUF

                @pl.when(jnp.logical_and(j >= NBUF - GDEPTH, j + GDEPTH < K))
                def _():
                    wait_write(j + GDEPTH - NBUF, bg)

                @pl.when(j + GDEPTH < K)
                def _():
                    gather(j + GDEPTH, bg)

                wait_gather(j, b)
                fixup(j, bufs[b])
                write(j, b)
            return 0

        lax.fori_loop(0, K // NBUF, step, 0)

        for j in range(K - NBUF, K):
            wait_write(j, j % NBUF)

    return k(ids3, mask3, table)


def kernel(input_ids, attention_mask, table):
    B, S = input_ids.shape
    n = B * S
    K = n // (NW * CHUNK)
    ids3 = input_ids.reshape(NW, K, CHUNK)
    mask3 = attention_mask.astype(jnp.float32).reshape(NW, K, CHUNK)
    out = _sc_embed(ids3, mask3, table)
    return out.reshape(B, S, D)
